# Initial kernel scaffold; baseline (speedup 1.0000x reference)
import jax
import jax.numpy as jnp
from jax.experimental import pallas as pl


def kernel(table, conv_w, conv_b, input_ids):
    raise NotImplementedError("write your pallas kernel here")


# speedup vs baseline: 1.7911x; 1.7911x over previous
import functools

import jax
import jax.numpy as jnp
from jax import lax
from jax.experimental import pallas as pl
from jax.experimental.pallas import tpu as pltpu
from jax.experimental.pallas import tpu_sc as plsc


def _t2_matmul(table, wcat, bias2, V, D, F):
    def body(a_ref, b_ref, bias_ref, o_ref):
        acc = jnp.dot(a_ref[...], b_ref[...], preferred_element_type=jnp.float32)
        sel = (pl.program_id(0) == 0).astype(jnp.float32)
        o_ref[...] = acc + bias_ref[...] * sel

    return pl.pallas_call(
        body,
        grid=(F,),
        in_specs=[
            pl.BlockSpec((V, D), lambda k: (0, 0)),
            pl.BlockSpec((D, D), lambda k: (0, k)),
            pl.BlockSpec((1, D), lambda k: (0, 0)),
        ],
        out_specs=pl.BlockSpec((V, D), lambda k: (0, k)),
        out_shape=jax.ShapeDtypeStruct((V, F * D), jnp.float32),
    )(table, wcat, bias2)


def _sc_gather_sum(t2r, ids3, R, D, F, NC, NS):
    NW = NC * NS
    rows_w = R // NW
    CH = 16
    CHUNKS = rows_w // CH
    GR = CH * F
    mesh = plsc.VectorSubcoreMesh(core_axis_name="c", subcore_axis_name="s")

    @functools.partial(
        pl.kernel,
        mesh=mesh,
        out_type=jax.ShapeDtypeStruct((R, D), jnp.float32),
        scratch_types=[
            pltpu.VMEM((CHUNKS, GR), jnp.int32),
            pltpu.VMEM((GR, D), jnp.float32),
            pltpu.VMEM((CH, D), jnp.float32),
            pltpu.SemaphoreType.DMA,
        ],
    )
    def k(t2_hbm, ids_hbm, out_hbm, idx_v, rows_v, acc_v, sem):
        wid = lax.axis_index("s") * NC + lax.axis_index("c")
        pltpu.sync_copy(ids_hbm.at[wid], idx_v)
        pat = lax.rem(lax.iota(jnp.int32, 16), jnp.full((16,), F, jnp.int32))
        for c in range(CHUNKS):
            for i in range(GR // 16):
                sl = pl.ds(i * 16, 16)
                idx_v[c, sl] = idx_v[c, sl] * F + pat

        nsl = D // 16
        for c in range(CHUNKS):
            pltpu.async_copy(t2_hbm.at[idx_v.at[c]], rows_v, sem).wait()

            def accbody(t, carry):
                j = t // nsl
                q = t - j * nsl
                sl = pl.ds(q * 16, 16)
                acc_v[j, sl] = (rows_v[F * j, sl] + rows_v[F * j + 1, sl]) + (
                    rows_v[F * j + 2, sl] + rows_v[F * j + 3, sl]
                )
                return carry

            lax.fori_loop(0, CH * nsl, accbody, 0, unroll=8)
            pltpu.sync_copy(acc_v, out_hbm.at[pl.ds(wid * rows_w + c * CH, CH)])

    return k(t2r, ids3)


def kernel(table, conv_w, conv_b, input_ids):
    V, D = table.shape
    F = conv_w.shape[2]
    B, S = input_ids.shape
    R = B * S // F

    info = plsc.get_sparse_core_info()
    NC, NS = info.num_cores, info.num_subcores
    NW = NC * NS

    wcat = conv_w.transpose(1, 2, 0).reshape(D, F * D)
    t2 = _t2_matmul(table, wcat, conv_b.reshape(1, D), V, D, F)
    t2r = t2.reshape(V * F, D)

    rows_w = R // NW
    CH = 16
    ids3 = input_ids.reshape(NW, rows_w // CH, CH * F).astype(jnp.int32)
    out_flat = _sc_gather_sum(t2r, ids3, R, D, F, NC, NS)
    return out_flat.reshape(B, S // F, D)


# speedup vs baseline: 2.4392x; 1.3618x over previous
import functools

import jax
import jax.numpy as jnp
from jax import lax
from jax.experimental import pallas as pl
from jax.experimental.pallas import tpu as pltpu
from jax.experimental.pallas import tpu_sc as plsc


def _t2_matmul(table, wcat, bias2, V, D, F):
    def body(a_ref, b_ref, bias_ref, o_ref):
        acc = jnp.dot(a_ref[...], b_ref[...], preferred_element_type=jnp.float32)
        sel = (pl.program_id(0) == 0).astype(jnp.float32)
        o_ref[...] = acc + bias_ref[...] * sel

    return pl.pallas_call(
        body,
        grid=(F,),
        in_specs=[
            pl.BlockSpec((V, D), lambda k: (0, 0)),
            pl.BlockSpec((D, D), lambda k: (0, k)),
            pl.BlockSpec((1, D), lambda k: (0, 0)),
        ],
        out_specs=pl.BlockSpec((V, D), lambda k: (k, 0)),
        out_shape=jax.ShapeDtypeStruct((F * V, D), jnp.float32),
    )(table, wcat, bias2)


def _sc_gather_sum(t2r, ids3, R, V, D, F, NC, NS):
    NW = NC * NS
    rows_w = R // NW
    CH = 8
    CHUNKS = rows_w // CH
    GR = CH * F
    mesh = plsc.VectorSubcoreMesh(core_axis_name="c", subcore_axis_name="s")

    @functools.partial(
        pl.kernel,
        mesh=mesh,
        out_type=jax.ShapeDtypeStruct((R, D), jnp.float32),
        scratch_types=[
            pltpu.VMEM((CHUNKS, GR), jnp.int32),
            pltpu.VMEM((2, GR, D), jnp.float32),
            pltpu.VMEM((2, CH, D), jnp.float32),
            pltpu.SemaphoreType.DMA,
            pltpu.SemaphoreType.DMA,
            pltpu.SemaphoreType.DMA,
        ],
    )
    def k(t2_hbm, ids_hbm, out_hbm, idx_v, rows_v, acc_v, gsem0, gsem1, osem):
        wid = lax.axis_index("s") * NC + lax.axis_index("c")
        pltpu.sync_copy(ids_hbm.at[wid], idx_v)
        pat = lax.rem(lax.iota(jnp.int32, 16), jnp.full((16,), F, jnp.int32))
        patv = pat * V
        for c in range(CHUNKS):
            for i in range(GR // 16):
                sl = pl.ds(i * 16, 16)
                idx_v[c, sl] = idx_v[c, sl] + patv

        gsems = (gsem0, gsem1)
        nsl = D // 16
        base = wid * rows_w

        def start_gather(c):
            return pltpu.async_copy(
                t2_hbm.at[idx_v.at[c]], rows_v.at[c % 2], gsems[c % 2]
            )

        pend = start_gather(0)
        outh = [None, None]
        for c in range(CHUNKS):
            pend.wait()
            if c + 1 < CHUNKS:
                pend = start_gather(c + 1)
            rb = rows_v.at[c % 2]
            ab = acc_v.at[c % 2]
            if outh[c % 2] is not None:
                outh[c % 2].wait()

            def accbody(t, carry):
                j = t // nsl
                q = t - j * nsl
                sl = pl.ds(q * 16, 16)
                ab[j, sl] = (rb[F * j, sl] + rb[F * j + 1, sl]) + (
                    rb[F * j + 2, sl] + rb[F * j + 3, sl]
                )
                return carry

            lax.fori_loop(0, CH * nsl, accbody, 0, unroll=8)
            outh[c % 2] = pltpu.async_copy(
                ab, out_hbm.at[pl.ds(base + c * CH, CH)], osem
            )
        outh[(CHUNKS - 1) % 2].wait()
        outh[CHUNKS % 2].wait()

    return k(t2r, ids3)


def kernel(table, conv_w, conv_b, input_ids):
    V, D = table.shape
    F = conv_w.shape[2]
    B, S = input_ids.shape
    R = B * S // F

    info = plsc.get_sparse_core_info()
    NC, NS = info.num_cores, info.num_subcores
    NW = NC * NS

    wcat = conv_w.transpose(1, 2, 0).reshape(D, F * D)
    t2r = _t2_matmul(table, wcat, conv_b.reshape(1, D), V, D, F)

    rows_w = R // NW
    CH = 8
    ids3 = input_ids.reshape(NW, rows_w // CH, CH * F).astype(jnp.int32)
    out_flat = _sc_gather_sum(t2r, ids3, R, V, D, F, NC, NS)
    return out_flat.reshape(B, S // F, D)
